# baseline (speedup 1.0000x reference)
import jax
import jax.numpy as jnp
from jax.experimental import pallas as pl
from jax.experimental.pallas import tpu as pltpu

_L, _B, _H, _S = 12, 4, 12, 197
_BH = _B * _H
_G = 8
_K = 6
_KPAD = 8


def _chain_topk_body(x_ref, attn_ref, idx_ref):
    ms = [x_ref[0, g] for g in range(_G)]
    for layer in range(1, _L - 1):
        xl = x_ref[layer]
        for g in range(_G):
            ms[g] = jax.lax.dot_general(
                xl[g], ms[g],
                (((1,), (0,)), ((), ())),
                preferred_element_type=jnp.float32).astype(jnp.bfloat16)
    rows = []
    for g in range(_G):
        v = jax.lax.dot_general(
            x_ref[_L - 1, g, 0:1, :], ms[g],
            (((1,), (0,)), ((), ())),
            preferred_element_type=jnp.float32)
        rows.append(v)
    vals = jnp.concatenate(rows, axis=0)

    lane = jax.lax.broadcasted_iota(jnp.int32, (_G, _S), 1)
    big = jnp.int32(10 ** 9)
    neg = jnp.float32(-jnp.inf)
    vals = jnp.where(lane == 0, neg, vals)

    lane8 = jax.lax.broadcasted_iota(jnp.int32, (_G, _KPAD), 1)
    attn_v = jnp.zeros((_G, _KPAD), jnp.float32)
    idx_v = jnp.zeros((_G, _KPAD), jnp.int32)
    for step in range(_K):
        m = jnp.max(vals, axis=1, keepdims=True)
        hit = vals == m
        idx = jnp.min(jnp.where(hit, lane, big), axis=1, keepdims=True)
        attn_v = jnp.where(lane8 == step, m, attn_v)
        idx_v = jnp.where(lane8 == step, idx - 1, idx_v)
        vals = jnp.where(lane == idx, neg, vals)
    attn_ref[0] = attn_v
    idx_ref[0] = idx_v


def _run_chain_topk(xr, interpret=False):
    nsteps = _BH // _G
    return pl.pallas_call(
        _chain_topk_body,
        grid=(nsteps,),
        in_specs=[
            pl.BlockSpec((_L, _G, _S, _S), lambda i: (0, i, 0, 0)),
        ],
        out_specs=[
            pl.BlockSpec((1, _G, _KPAD), lambda i: (i, 0, 0)),
            pl.BlockSpec((1, _G, _KPAD), lambda i: (i, 0, 0)),
        ],
        out_shape=[
            jax.ShapeDtypeStruct((nsteps, _G, _KPAD), jnp.float32),
            jax.ShapeDtypeStruct((nsteps, _G, _KPAD), jnp.int32),
        ],
        interpret=interpret,
    )(xr)


def kernel(x, k):
    xt = jnp.transpose(x.astype(jnp.bfloat16), (0, 2, 1, 3, 4))
    xr = xt.reshape(_L, _BH, _S, _S)
    attn, idx = _run_chain_topk(xr)
    attn_hb = attn.reshape(_H, _B, _KPAD)[:, :, :_K]
    idx_hb = idx.reshape(_H, _B, _KPAD)[:, :, :_K]
    max_attn = jnp.transpose(attn_hb, (1, 0, 2)).reshape(_B, _H * _K)
    max_inx = jnp.transpose(idx_hb, (1, 0, 2)).reshape(_B, _H * _K)
    k_zero = jnp.asarray(k) - jnp.asarray(k)
    max_attn = max_attn + k_zero.astype(max_attn.dtype)
    max_inx = max_inx + k_zero.astype(max_inx.dtype)
    return (max_attn, max_inx)


# speedup vs baseline: 1.2373x; 1.2373x over previous
import jax
import jax.numpy as jnp
from jax.experimental import pallas as pl
from jax.experimental.pallas import tpu as pltpu

_L, _B, _H, _S = 12, 4, 12, 197
_BH = _B * _H
_G = 8
_K = 6
_KPAD = 8


def _chain_topk_body(*refs):
    layer_refs = refs[:_L - 1]
    v0_ref = refs[_L - 1]
    attn_ref, idx_ref = refs[_L], refs[_L + 1]
    ms = [layer_refs[0][g] for g in range(_G)]
    for layer in range(1, _L - 1):
        xl = layer_refs[layer][...]
        for g in range(_G):
            ms[g] = jax.lax.dot_general(
                xl[g], ms[g],
                (((1,), (0,)), ((), ())),
                preferred_element_type=jnp.float32).astype(jnp.bfloat16)
    rows = []
    for g in range(_G):
        v = jax.lax.dot_general(
            v0_ref[g], ms[g],
            (((1,), (0,)), ((), ())),
            preferred_element_type=jnp.float32)
        rows.append(v)
    vals = jnp.concatenate(rows, axis=0)

    lane = jax.lax.broadcasted_iota(jnp.int32, (_G, _S), 1)
    big = jnp.int32(10 ** 9)
    neg = jnp.float32(-jnp.inf)
    vals = jnp.where(lane == 0, neg, vals)

    lane8 = jax.lax.broadcasted_iota(jnp.int32, (_G, _KPAD), 1)
    attn_v = jnp.zeros((_G, _KPAD), jnp.float32)
    idx_v = jnp.zeros((_G, _KPAD), jnp.int32)
    for step in range(_K):
        m = jnp.max(vals, axis=1, keepdims=True)
        hit = vals == m
        idx = jnp.min(jnp.where(hit, lane, big), axis=1, keepdims=True)
        attn_v = jnp.where(lane8 == step, m, attn_v)
        idx_v = jnp.where(lane8 == step, idx - 1, idx_v)
        vals = jnp.where(lane == idx, neg, vals)
    attn_ref[0] = attn_v
    idx_ref[0] = idx_v


def _run_chain_topk(layers, v0, interpret=False):
    nsteps = _BH // _G
    return pl.pallas_call(
        _chain_topk_body,
        grid=(nsteps,),
        in_specs=(
            [pl.BlockSpec((_G, _S, _S), lambda i: (i, 0, 0))] * (_L - 1)
            + [pl.BlockSpec((_G, 1, _S), lambda i: (i, 0, 0))]
        ),
        out_specs=[
            pl.BlockSpec((1, _G, _KPAD), lambda i: (i, 0, 0)),
            pl.BlockSpec((1, _G, _KPAD), lambda i: (i, 0, 0)),
        ],
        out_shape=[
            jax.ShapeDtypeStruct((nsteps, _G, _KPAD), jnp.float32),
            jax.ShapeDtypeStruct((nsteps, _G, _KPAD), jnp.int32),
        ],
        interpret=interpret,
    )(*layers, v0)


def kernel(x, k):
    layers = [x[i].astype(jnp.bfloat16).reshape(_BH, _S, _S)
              for i in range(_L - 1)]
    v0 = x[_L - 1, :, :, 0:1, :].astype(jnp.bfloat16).reshape(_BH, 1, _S)
    attn, idx = _run_chain_topk(layers, v0)
    max_attn = attn.reshape(_BH, _KPAD)[:, :_K].reshape(_B, _H * _K)
    max_inx = idx.reshape(_BH, _KPAD)[:, :_K].reshape(_B, _H * _K)
    k_zero = jnp.asarray(k) - jnp.asarray(k)
    max_attn = max_attn + k_zero.astype(max_attn.dtype)
    max_inx = max_inx + k_zero.astype(max_inx.dtype)
    return (max_attn, max_inx)
